# baseline (speedup 1.0000x reference)
import jax
import jax.numpy as jnp
from jax import lax
from jax.experimental import pallas as pl
from jax.experimental.pallas import tpu as pltpu
from jax.experimental.pallas import tpu_sc as plsc

_B = 16384
_T = 24
_D = 128
_NC, _NS = 2, 16
_NW = _NC * _NS
_ROWS_PER_W = _B * _T // _NW
_NCHUNK = _ROWS_PER_W // _T


def _sc_body(tab_hbm, out_hbm, tab_v, sem):
    wid = lax.axis_index("c") * _NS + lax.axis_index("s")
    base = wid * _ROWS_PER_W
    stage = [
        pltpu.async_copy(tab_hbm, tab_v.at[pl.ds(k * _T, _T)], sem)
        for k in range(2)
    ]
    for c in stage:
        c.wait()
    copies = [
        pltpu.async_copy(
            tab_v.at[pl.ds((j % 2) * _T, _T)],
            out_hbm.at[pl.ds(base + j * _T, _T)],
            sem,
        )
        for j in range(_NCHUNK)
    ]
    for c in copies:
        c.wait()


def kernel(t_future, batch_size, table):
    del t_future, batch_size
    k = pl.kernel(
        _sc_body,
        out_type=jax.ShapeDtypeStruct((_B * _T, _D), jnp.float32),
        scratch_types=[
            pltpu.VMEM((2 * _T, _D), jnp.float32),
            pltpu.SemaphoreType.DMA,
        ],
        mesh=plsc.VectorSubcoreMesh(core_axis_name="c", subcore_axis_name="s"),
    )
    return k(table).reshape(_B, _T, _D)


# speedup vs baseline: 1.0590x; 1.0590x over previous
import jax
import jax.numpy as jnp
from jax import lax
from jax.experimental import pallas as pl
from jax.experimental.pallas import tpu as pltpu
from jax.experimental.pallas import tpu_sc as plsc

_B = 16384
_T = 24
_D = 128
_NC, _NS = 2, 16
_NW = _NC * _NS
_ROWS_PER_W = _B * _T // _NW
_NCHUNK = _ROWS_PER_W // _T


def _sc_body(tab_hbm, out_hbm, tab_v, sem):
    wid = lax.axis_index("c") * _NS + lax.axis_index("s")
    base = wid * _ROWS_PER_W
    pltpu.async_copy(tab_hbm, tab_v, sem).wait()
    copies = [
        pltpu.async_copy(tab_v, out_hbm.at[pl.ds(base + j * _T, _T)], sem)
        for j in range(_NCHUNK)
    ]
    for c in copies:
        c.wait()


def kernel(t_future, batch_size, table):
    del t_future, batch_size
    k = pl.kernel(
        _sc_body,
        out_type=jax.ShapeDtypeStruct((_B * _T, _D), jnp.float32),
        scratch_types=[
            pltpu.VMEM((_T, _D), jnp.float32),
            pltpu.SemaphoreType.DMA,
        ],
        mesh=plsc.VectorSubcoreMesh(core_axis_name="c", subcore_axis_name="s"),
    )
    return k(table).reshape(_B, _T, _D)
